# baseline (speedup 1.0000x reference)
import functools

import jax
import jax.numpy as jnp
from jax import lax
from jax.experimental import pallas as pl
from jax.experimental.pallas import tpu as pltpu
from jax.experimental.pallas import tpu_sc as plsc

N = 2048
E = 65536
NTYPE = 4
NMAT = 4
NC, NS, L = 2, 16, 16
RP = 256
NPASS = N // (RP * NC)
CHUNK = E // NS
BUF = RP * N
SLICE = BUF // NS
SCAT = 512
CAP = CHUNK + SCAT


@functools.partial(
    pl.kernel,
    out_type=jax.ShapeDtypeStruct((NTYPE, N * N), jnp.float32),
    mesh=plsc.VectorSubcoreMesh(core_axis_name="c", subcore_axis_name="s"),
    compiler_params=pltpu.CompilerParams(needs_layout_passes=False),
    scratch_types=[
        pltpu.VMEM((NTYPE * CHUNK,), jnp.int32),
        pltpu.VMEM((NTYPE * CHUNK,), jnp.int32),
        pltpu.VMEM((NTYPE * CHUNK,), jnp.float32),
        pltpu.VMEM((CAP,), jnp.int32),
        pltpu.VMEM((CAP,), jnp.float32),
        pltpu.VMEM_SHARED((BUF,), jnp.float32),
        pltpu.VMEM_SHARED((BUF,), jnp.float32),
        pltpu.SemaphoreType.DMA,
        pltpu.SemaphoreType.DMA,
        pltpu.SemaphoreType.DMA,
        pltpu.SemaphoreType.DMA,
    ],
)
def _sc_scatter(ei0, ei1, ei2, ei3, ev0, ev1, ev2, ev3, z_hbm, d_hbm,
                src_v, dst_v, val_v, idx_b, sval_b,
                sh0, sh1, esem, zsem, wsem0, wsem1):
    core = lax.axis_index("c")
    sub = lax.axis_index("s")

    off = sub * CHUNK
    edescs = []
    for j, (ei, ev) in enumerate(((ei0, ev0), (ei1, ev1), (ei2, ev2),
                                  (ei3, ev3))):
        edescs.append(pltpu.async_copy(
            ei.at[0].at[pl.ds(off, CHUNK)],
            src_v.at[pl.ds(j * CHUNK, CHUNK)], esem))
        edescs.append(pltpu.async_copy(
            ei.at[1].at[pl.ds(off, CHUNK)],
            dst_v.at[pl.ds(j * CHUNK, CHUNK)], esem))
        edescs.append(pltpu.async_copy(
            ev.at[pl.ds(off, CHUNK)],
            val_v.at[pl.ds(j * CHUNK, CHUNK)], esem))
    for dsc in edescs:
        dsc.wait()

    lanes = lax.iota(jnp.int32, L)
    zi = jnp.zeros((L,), jnp.int32)
    zf = jnp.zeros((L,), jnp.float32)
    shbufs = (sh0, sh1)
    wsems = (wsem0, wsem1)
    wdesc = [None, None]

    passes = [(j, p) for j in range(NTYPE) for p in range(NPASS)]
    zdesc = pltpu.async_copy(
        z_hbm, shbufs[0].at[pl.ds(sub * SLICE, SLICE)], zsem)
    for q, (j, p) in enumerate(passes):
            b = q % 2
            shb = shbufs[b]

            def _scan(g, pos):
                for u in range(2):
                    o = pl.multiple_of(j * CHUNK + (g * 2 + u) * L, L)
                    s16 = src_v[pl.ds(o, L)]
                    d16 = dst_v[pl.ds(o, L)]
                    v16 = val_v[pl.ds(o, L)]
                    grp = lax.shift_right_logical(s16, 8)
                    m = grp == (p * NC + core)
                    idx = (s16 & (RP - 1)) * N + d16
                    plsc.store_compressed(idx_b.at[pl.ds(pos, L)], idx, mask=m)
                    plsc.store_compressed(sval_b.at[pl.ds(pos, L)], v16,
                                          mask=m)
                    pos = pos + plsc.all_reduce_population_count(m)[0]
                return pos

            pos = lax.fori_loop(0, CHUNK // L // 2, _scan, 0)

            for k in range(SCAT // L):
                tgt = pos + k * L + lanes
                plsc.store_scatter(idx_b, [tgt], zi)
                plsc.store_scatter(sval_b, [tgt], zf)

            zdesc.wait()
            plsc.subcore_barrier()

            def _scat(t, carry):
                o = pl.multiple_of(t * SCAT, L)
                pltpu.sync_copy(sval_b.at[pl.ds(o, SCAT)],
                                shb.at[idx_b.at[pl.ds(o, SCAT)]],
                                add=True)
                return carry

            nch = (pos + SCAT - 1) // SCAT
            lax.fori_loop(0, nch, _scat, 0)
            plsc.subcore_barrier()

            dst0 = (p * NC + core) * BUF + sub * SLICE
            wdesc[b] = pltpu.async_copy(
                shb.at[pl.ds(sub * SLICE, SLICE)],
                d_hbm.at[j].at[pl.ds(dst0, SLICE)], wsems[b])

            if q + 1 < len(passes):
                bn = (q + 1) % 2
                if wdesc[bn] is not None:
                    wdesc[bn].wait()
                zdesc = pltpu.async_copy(
                    z_hbm, shbufs[bn].at[pl.ds(sub * SLICE, SLICE)], zsem)

    wdesc[0].wait()
    wdesc[1].wait()


_BR = 128


def _softmax_body(c1_ref, c2_ref, f1_ref, f2_ref, w_ref):
    x1 = c1_ref[...]
    x2 = c2_ref[...]
    e1 = jnp.exp(x1 - jnp.max(x1, axis=1, keepdims=True))
    f1 = e1 / jnp.sum(e1, axis=1, keepdims=True)
    e2 = jnp.exp(x2 - jnp.max(x2, axis=1, keepdims=True))
    f2 = e2 / jnp.sum(e2, axis=1, keepdims=True)
    f1_ref[...] = f1
    f2_ref[...] = f2
    w_ref[...] = jnp.concatenate([f1, f2], axis=0)


_softmax_call = pl.pallas_call(
    _softmax_body,
    out_shape=(
        jax.ShapeDtypeStruct((2, NTYPE), jnp.float32),
        jax.ShapeDtypeStruct((2, NTYPE), jnp.float32),
        jax.ShapeDtypeStruct((NMAT, NTYPE), jnp.float32),
    ),
)


def _mix_body(w_ref, d_ref, ab_ref):
    for c in range(NMAT):
        acc = w_ref[c, 0] * d_ref[0].reshape(_BR, N)
        for j in range(1, NTYPE):
            acc += w_ref[c, j] * d_ref[j].reshape(_BR, N)
        ab_ref[c] = acc


_mix_call = pl.pallas_call(
    _mix_body,
    grid=(N // _BR,),
    in_specs=[
        pl.BlockSpec(memory_space=pltpu.SMEM),
        pl.BlockSpec((NTYPE, _BR * N), lambda r: (0, r)),
    ],
    out_specs=pl.BlockSpec((NMAT, _BR, N), lambda r: (0, r, 0)),
    out_shape=jax.ShapeDtypeStruct((NMAT, N, N), jnp.float32),
)


_BM = 512


def _mm_body(a_ref, b_ref, h_ref):
    h_ref[0] = jnp.dot(a_ref[0], b_ref[0],
                       preferred_element_type=jnp.float32)


_mm_call = pl.pallas_call(
    _mm_body,
    grid=(2, N // _BM),
    in_specs=[
        pl.BlockSpec((1, _BM, N), lambda i, mi: (i, mi, 0)),
        pl.BlockSpec((1, N, N), lambda i, mi: (i + 2, 0, 0)),
    ],
    out_specs=pl.BlockSpec((1, _BM, N), lambda i, mi: (i, mi, 0)),
    out_shape=jax.ShapeDtypeStruct((2, N, N), jnp.float32),
)


def kernel(edge_index_0, edge_index_1, edge_index_2, edge_index_3,
           edge_value_0, edge_value_1, edge_value_2, edge_value_3,
           conv1_weight, conv2_weight):
    f1, f2, w = _softmax_call(conv1_weight, conv2_weight)
    zeros = jnp.zeros((SLICE,), jnp.float32)
    d = _sc_scatter(edge_index_0, edge_index_1, edge_index_2, edge_index_3,
                    edge_value_0, edge_value_1, edge_value_2, edge_value_3,
                    zeros)
    ab = _mix_call(w, d)
    h = _mm_call(ab, ab)
    return (h, lax.stop_gradient(f1), lax.stop_gradient(f2))


# speedup vs baseline: 1.1511x; 1.1511x over previous
import functools

import jax
import jax.numpy as jnp
from jax import lax
from jax.experimental import pallas as pl
from jax.experimental.pallas import tpu as pltpu
from jax.experimental.pallas import tpu_sc as plsc

N = 2048
E = 65536
NTYPE = 4
NMAT = 4
NC, NS, L = 2, 16, 16
RP = 256
NPASS = N // (RP * NC)
CHUNK = E // NS
BUF = RP * N
SLICE = BUF // NS
SCAT = 256
CAP = CHUNK + SCAT


@functools.partial(
    pl.kernel,
    out_type=jax.ShapeDtypeStruct((NTYPE, N * N), jnp.float32),
    mesh=plsc.VectorSubcoreMesh(core_axis_name="c", subcore_axis_name="s"),
    compiler_params=pltpu.CompilerParams(needs_layout_passes=False),
    scratch_types=[
        pltpu.VMEM((NTYPE * CHUNK,), jnp.int32),
        pltpu.VMEM((NTYPE * CHUNK,), jnp.int32),
        pltpu.VMEM((NTYPE * CHUNK,), jnp.float32),
        pltpu.VMEM((CAP,), jnp.int32),
        pltpu.VMEM((CAP,), jnp.float32),
        pltpu.VMEM_SHARED((BUF,), jnp.float32),
        pltpu.VMEM_SHARED((BUF,), jnp.float32),
        pltpu.SemaphoreType.DMA,
        pltpu.SemaphoreType.DMA,
        pltpu.SemaphoreType.DMA,
        pltpu.SemaphoreType.DMA,
    ],
)
def _sc_scatter(ei0, ei1, ei2, ei3, ev0, ev1, ev2, ev3, z_hbm, d_hbm,
                src_v, dst_v, val_v, idx_b, sval_b,
                sh0, sh1, esem, zsem, wsem0, wsem1):
    core = lax.axis_index("c")
    sub = lax.axis_index("s")

    off = sub * CHUNK
    edescs = []
    for j, (ei, ev) in enumerate(((ei0, ev0), (ei1, ev1), (ei2, ev2),
                                  (ei3, ev3))):
        edescs.append(pltpu.async_copy(
            ei.at[0].at[pl.ds(off, CHUNK)],
            src_v.at[pl.ds(j * CHUNK, CHUNK)], esem))
        edescs.append(pltpu.async_copy(
            ei.at[1].at[pl.ds(off, CHUNK)],
            dst_v.at[pl.ds(j * CHUNK, CHUNK)], esem))
        edescs.append(pltpu.async_copy(
            ev.at[pl.ds(off, CHUNK)],
            val_v.at[pl.ds(j * CHUNK, CHUNK)], esem))
    for dsc in edescs:
        dsc.wait()

    lanes = lax.iota(jnp.int32, L)
    zi = jnp.zeros((L,), jnp.int32)
    zf = jnp.zeros((L,), jnp.float32)
    shbufs = (sh0, sh1)
    wsems = (wsem0, wsem1)
    wdesc = [None, None]

    passes = [(j, p) for j in range(NTYPE) for p in range(NPASS)]
    zdesc = pltpu.async_copy(
        z_hbm, shbufs[0].at[pl.ds(sub * SLICE, SLICE)], zsem)
    for q, (j, p) in enumerate(passes):
            b = q % 2
            shb = shbufs[b]

            def _scan(g, pos):
                o = pl.multiple_of(j * CHUNK + g * L, L)
                s16 = src_v[pl.ds(o, L)]
                d16 = dst_v[pl.ds(o, L)]
                v16 = val_v[pl.ds(o, L)]
                grp = lax.shift_right_logical(s16, 8)
                m = grp == (p * NC + core)
                idx = (s16 & (RP - 1)) * N + d16
                plsc.store_compressed(idx_b.at[pl.ds(pos, L)], idx, mask=m)
                plsc.store_compressed(sval_b.at[pl.ds(pos, L)], v16, mask=m)
                return pos + plsc.all_reduce_population_count(m)[0]

            pos = lax.fori_loop(0, CHUNK // L, _scan, 0)

            for k in range(SCAT // L):
                tgt = pos + k * L + lanes
                plsc.store_scatter(idx_b, [tgt], zi)
                plsc.store_scatter(sval_b, [tgt], zf)

            zdesc.wait()
            plsc.subcore_barrier()

            def _scat(t, carry):
                o = pl.multiple_of(t * SCAT, L)
                pltpu.sync_copy(sval_b.at[pl.ds(o, SCAT)],
                                shb.at[idx_b.at[pl.ds(o, SCAT)]],
                                add=True)
                return carry

            nch = (pos + SCAT - 1) // SCAT
            lax.fori_loop(0, nch, _scat, 0)
            plsc.subcore_barrier()

            dst0 = (p * NC + core) * BUF + sub * SLICE
            wdesc[b] = pltpu.async_copy(
                shb.at[pl.ds(sub * SLICE, SLICE)],
                d_hbm.at[j].at[pl.ds(dst0, SLICE)], wsems[b])

            if q + 1 < len(passes):
                bn = (q + 1) % 2
                if wdesc[bn] is not None:
                    wdesc[bn].wait()
                zdesc = pltpu.async_copy(
                    z_hbm, shbufs[bn].at[pl.ds(sub * SLICE, SLICE)], zsem)

    wdesc[0].wait()
    wdesc[1].wait()


_BR = 128


def _softmax_body(c1_ref, c2_ref, f1_ref, f2_ref, w_ref):
    x1 = c1_ref[...]
    x2 = c2_ref[...]
    e1 = jnp.exp(x1 - jnp.max(x1, axis=1, keepdims=True))
    f1 = e1 / jnp.sum(e1, axis=1, keepdims=True)
    e2 = jnp.exp(x2 - jnp.max(x2, axis=1, keepdims=True))
    f2 = e2 / jnp.sum(e2, axis=1, keepdims=True)
    f1_ref[...] = f1
    f2_ref[...] = f2
    w_ref[...] = jnp.concatenate([f1, f2], axis=0)


_softmax_call = pl.pallas_call(
    _softmax_body,
    out_shape=(
        jax.ShapeDtypeStruct((2, NTYPE), jnp.float32),
        jax.ShapeDtypeStruct((2, NTYPE), jnp.float32),
        jax.ShapeDtypeStruct((NMAT, NTYPE), jnp.float32),
    ),
)


def _mix_body(w_ref, d_ref, ab_ref):
    for c in range(NMAT):
        acc = w_ref[c, 0] * d_ref[0].reshape(_BR, N)
        for j in range(1, NTYPE):
            acc += w_ref[c, j] * d_ref[j].reshape(_BR, N)
        ab_ref[c] = acc


_mix_call = pl.pallas_call(
    _mix_body,
    grid=(N // _BR,),
    in_specs=[
        pl.BlockSpec(memory_space=pltpu.SMEM),
        pl.BlockSpec((NTYPE, _BR * N), lambda r: (0, r)),
    ],
    out_specs=pl.BlockSpec((NMAT, _BR, N), lambda r: (0, r, 0)),
    out_shape=jax.ShapeDtypeStruct((NMAT, N, N), jnp.float32),
)


_BM = 512


def _mm_body(a_ref, b_ref, h_ref):
    h_ref[0] = jnp.dot(a_ref[0], b_ref[0],
                       preferred_element_type=jnp.float32)


_mm_call = pl.pallas_call(
    _mm_body,
    grid=(2, N // _BM),
    in_specs=[
        pl.BlockSpec((1, _BM, N), lambda i, mi: (i, mi, 0)),
        pl.BlockSpec((1, N, N), lambda i, mi: (i + 2, 0, 0)),
    ],
    out_specs=pl.BlockSpec((1, _BM, N), lambda i, mi: (i, mi, 0)),
    out_shape=jax.ShapeDtypeStruct((2, N, N), jnp.float32),
)


def kernel(edge_index_0, edge_index_1, edge_index_2, edge_index_3,
           edge_value_0, edge_value_1, edge_value_2, edge_value_3,
           conv1_weight, conv2_weight):
    f1, f2, w = _softmax_call(conv1_weight, conv2_weight)
    zeros = jnp.zeros((SLICE,), jnp.float32)
    d = _sc_scatter(edge_index_0, edge_index_1, edge_index_2, edge_index_3,
                    edge_value_0, edge_value_1, edge_value_2, edge_value_3,
                    zeros)
    ab = _mix_call(w, d)
    h = _mm_call(ab, ab)
    return (h, lax.stop_gradient(f1), lax.stop_gradient(f2))


# speedup vs baseline: 1.2193x; 1.0592x over previous
import functools

import jax
import jax.numpy as jnp
from jax import lax
from jax.experimental import pallas as pl
from jax.experimental.pallas import tpu as pltpu
from jax.experimental.pallas import tpu_sc as plsc

N = 2048
E = 65536
NTYPE = 4
NMAT = 4
NC, NS, L = 2, 16, 16
RP = 256
NPASS = N // (RP * NC)
CHUNK = E // NS
BUF = RP * N
SLICE = BUF // NS
SCAT = 256
CAP = CHUNK + SCAT


@functools.partial(
    pl.kernel,
    out_type=jax.ShapeDtypeStruct((NTYPE, N * N), jnp.float32),
    mesh=plsc.VectorSubcoreMesh(core_axis_name="c", subcore_axis_name="s"),
    compiler_params=pltpu.CompilerParams(needs_layout_passes=False),
    scratch_types=[
        pltpu.VMEM((NTYPE * CHUNK,), jnp.int32),
        pltpu.VMEM((NTYPE * CHUNK,), jnp.int32),
        pltpu.VMEM((NTYPE * CHUNK,), jnp.float32),
        pltpu.VMEM((CAP,), jnp.int32),
        pltpu.VMEM((CAP,), jnp.float32),
        pltpu.VMEM_SHARED((BUF,), jnp.float32),
        pltpu.VMEM_SHARED((BUF,), jnp.float32),
        pltpu.SemaphoreType.DMA,
        pltpu.SemaphoreType.DMA,
        pltpu.SemaphoreType.DMA,
        pltpu.SemaphoreType.DMA,
    ],
)
def _sc_scatter(ei0, ei1, ei2, ei3, ev0, ev1, ev2, ev3, z_hbm, d_hbm,
                src_v, dst_v, val_v, idx_b, sval_b,
                sh0, sh1, esem, zsem, wsem0, wsem1):
    core = lax.axis_index("c")
    sub = lax.axis_index("s")

    off = sub * CHUNK
    edescs = []
    for j, (ei, ev) in enumerate(((ei0, ev0), (ei1, ev1), (ei2, ev2),
                                  (ei3, ev3))):
        edescs.append(pltpu.async_copy(
            ei.at[0].at[pl.ds(off, CHUNK)],
            src_v.at[pl.ds(j * CHUNK, CHUNK)], esem))
        edescs.append(pltpu.async_copy(
            ei.at[1].at[pl.ds(off, CHUNK)],
            dst_v.at[pl.ds(j * CHUNK, CHUNK)], esem))
        edescs.append(pltpu.async_copy(
            ev.at[pl.ds(off, CHUNK)],
            val_v.at[pl.ds(j * CHUNK, CHUNK)], esem))
    for dsc in edescs:
        dsc.wait()

    lanes = lax.iota(jnp.int32, L)
    zi = jnp.zeros((L,), jnp.int32)
    zf = jnp.zeros((L,), jnp.float32)
    shbufs = (sh0, sh1)
    wsems = (wsem0, wsem1)
    wdesc = [None, None]

    passes = [(j, p) for j in range(NTYPE) for p in range(NPASS)]
    zdesc = pltpu.async_copy(
        z_hbm, shbufs[0].at[pl.ds(sub * SLICE, SLICE)], zsem)
    for q, (j, p) in enumerate(passes):
            b = q % 2
            shb = shbufs[b]

            def _scan(g, pos):
                o = pl.multiple_of(j * CHUNK + g * L, L)
                s16 = src_v[pl.ds(o, L)]
                d16 = dst_v[pl.ds(o, L)]
                v16 = val_v[pl.ds(o, L)]
                grp = lax.shift_right_logical(s16, 8)
                m = grp == (p * NC + core)
                idx = (s16 & (RP - 1)) * N + d16
                plsc.store_compressed(idx_b.at[pl.ds(pos, L)], idx, mask=m)
                plsc.store_compressed(sval_b.at[pl.ds(pos, L)], v16, mask=m)
                return pos + plsc.all_reduce_population_count(m)[0]

            pos = lax.fori_loop(0, CHUNK // L, _scan, 0)

            for k in range(SCAT // L):
                tgt = pos + k * L + lanes
                plsc.store_scatter(idx_b, [tgt], zi)
                plsc.store_scatter(sval_b, [tgt], zf)

            zdesc.wait()
            plsc.subcore_barrier()

            def _scat(t, carry):
                o = pl.multiple_of(t * SCAT, L)
                pltpu.sync_copy(sval_b.at[pl.ds(o, SCAT)],
                                shb.at[idx_b.at[pl.ds(o, SCAT)]],
                                add=True)
                return carry

            nch = (pos + SCAT - 1) // SCAT
            lax.fori_loop(0, nch, _scat, 0)
            plsc.subcore_barrier()

            dst0 = (p * NC + core) * BUF + sub * SLICE
            wdesc[b] = pltpu.async_copy(
                shb.at[pl.ds(sub * SLICE, SLICE)],
                d_hbm.at[j].at[pl.ds(dst0, SLICE)], wsems[b])

            if q + 1 < len(passes):
                bn = (q + 1) % 2
                if wdesc[bn] is not None:
                    wdesc[bn].wait()
                zdesc = pltpu.async_copy(
                    z_hbm, shbufs[bn].at[pl.ds(sub * SLICE, SLICE)], zsem)

    wdesc[0].wait()
    wdesc[1].wait()


_BR = 128


def _softmax_body(c1_ref, c2_ref, f1_ref, f2_ref, w_ref):
    x1 = c1_ref[...]
    x2 = c2_ref[...]
    e1 = jnp.exp(x1 - jnp.max(x1, axis=1, keepdims=True))
    f1 = e1 / jnp.sum(e1, axis=1, keepdims=True)
    e2 = jnp.exp(x2 - jnp.max(x2, axis=1, keepdims=True))
    f2 = e2 / jnp.sum(e2, axis=1, keepdims=True)
    f1_ref[...] = f1
    f2_ref[...] = f2
    w_ref[...] = jnp.concatenate([f1, f2], axis=0)


_softmax_call = pl.pallas_call(
    _softmax_body,
    out_shape=(
        jax.ShapeDtypeStruct((2, NTYPE), jnp.float32),
        jax.ShapeDtypeStruct((2, NTYPE), jnp.float32),
        jax.ShapeDtypeStruct((NMAT, NTYPE), jnp.float32),
    ),
)


def _mix_body(w_ref, d_ref, ab_ref):
    for c in range(NMAT):
        acc = w_ref[c, 0] * d_ref[0].reshape(_BR, N)
        for j in range(1, NTYPE):
            acc += w_ref[c, j] * d_ref[j].reshape(_BR, N)
        ab_ref[c] = acc.astype(jnp.bfloat16)


_mix_call = pl.pallas_call(
    _mix_body,
    grid=(N // _BR,),
    in_specs=[
        pl.BlockSpec(memory_space=pltpu.SMEM),
        pl.BlockSpec((NTYPE, _BR * N), lambda r: (0, r)),
    ],
    out_specs=pl.BlockSpec((NMAT, _BR, N), lambda r: (0, r, 0)),
    out_shape=jax.ShapeDtypeStruct((NMAT, N, N), jnp.bfloat16),
)


_BM = 512


def _mm_body(a_ref, b_ref, h_ref):
    h_ref[0] = jnp.dot(a_ref[0], b_ref[0],
                       preferred_element_type=jnp.float32)


_mm_call = pl.pallas_call(
    _mm_body,
    grid=(2, N // _BM),
    in_specs=[
        pl.BlockSpec((1, _BM, N), lambda i, mi: (i, mi, 0)),
        pl.BlockSpec((1, N, N), lambda i, mi: (i + 2, 0, 0)),
    ],
    out_specs=pl.BlockSpec((1, _BM, N), lambda i, mi: (i, mi, 0)),
    out_shape=jax.ShapeDtypeStruct((2, N, N), jnp.float32),
)


def kernel(edge_index_0, edge_index_1, edge_index_2, edge_index_3,
           edge_value_0, edge_value_1, edge_value_2, edge_value_3,
           conv1_weight, conv2_weight):
    f1, f2, w = _softmax_call(conv1_weight, conv2_weight)
    zeros = jnp.zeros((SLICE,), jnp.float32)
    d = _sc_scatter(edge_index_0, edge_index_1, edge_index_2, edge_index_3,
                    edge_value_0, edge_value_1, edge_value_2, edge_value_3,
                    zeros)
    ab = _mix_call(w, d)
    h = _mm_call(ab, ab)
    return (h, lax.stop_gradient(f1), lax.stop_gradient(f2))
